# Initial kernel scaffold; baseline (speedup 1.0000x reference)
import jax
import jax.numpy as jnp
from jax.experimental import pallas as pl


def kernel(features, src_id, dst_id, W1, b1, W2, b2):
    raise NotImplementedError("write your pallas kernel here")


# speedup vs baseline: 2.3643x; 2.3643x over previous
import functools

import jax
import jax.numpy as jnp
from jax import lax
from jax.experimental import pallas as pl
from jax.experimental.pallas import tpu as pltpu
from jax.experimental.pallas import tpu_sc as plsc

N = 10000
E = 320000
C = 128
NCHUNK = 4
B = 128
NT = 16
EPT_ALIGN = NT * B
EP = ((E + EPT_ALIGN - 1) // EPT_ALIGN) * EPT_ALIGN
EPT = EP // NT
NB = EPT // B
RANGE = 5120
NACC = 5248
TRASH = 5200
ZROWS_A = NACC // NT
FROWS = RANGE // NT
NPASS = 8


def _stage1_body(f_ref, mb_ref, bb_ref, ms_ref, bs_ref, big_ref, small_ref):
    f = f_ref[...]
    big_ref[:N, :] = (
        jnp.dot(f, mb_ref[0].T, preferred_element_type=jnp.float32) + bb_ref[0]
    )
    big_ref[N:, :] = (
        jnp.dot(f, mb_ref[1].T, preferred_element_type=jnp.float32) + bb_ref[1]
    )
    small_ref[...] = (
        jnp.dot(f, ms_ref[0].T, preferred_element_type=jnp.float32) + bs_ref[0]
    )


def _build_tables(features, W1, b1):
    eye = jnp.eye(64, dtype=jnp.float32)
    zero = jnp.zeros((64, 64), jnp.float32)
    sel = [jnp.concatenate([eye, zero], axis=1),
           jnp.concatenate([zero, eye], axis=1)]
    mbig, bb, msm, bs = [], [], [], []
    for c in range(NCHUNK):
        rows = W1[64 * c:64 * (c + 1), :]
        b1c = b1[64 * c:64 * (c + 1)]
        if c < 2:
            wb, ws = rows[:, :C], rows[:, C:]
            bb.append(jnp.zeros((128,), jnp.float32))
            bs.append(b1c)
        else:
            wb, ws = rows[:, C:], rows[:, :C]
            bb.append(jnp.concatenate([b1c, jnp.zeros((64,), jnp.float32)]))
            bs.append(jnp.zeros((64,), jnp.float32))
        mbig.append(jnp.concatenate([wb, sel[c % 2]], axis=0))
        msm.append(ws)
    MB = jnp.stack(mbig)
    BB = jnp.stack(bb)[:, None, :]
    MS = jnp.stack([W1[0:128, C:], W1[C:, 0:C]])
    BS = jnp.stack([b1[0:C], jnp.zeros((C,), jnp.float32)])[:, None, :]

    return pl.pallas_call(
        _stage1_body,
        grid=(2,),
        in_specs=[
            pl.BlockSpec((N, C), lambda h: (0, 0)),
            pl.BlockSpec((2, 128, 128), lambda h: (h, 0, 0)),
            pl.BlockSpec((2, 1, 128), lambda h: (h, 0, 0)),
            pl.BlockSpec((1, 128, 128), lambda h: (h, 0, 0)),
            pl.BlockSpec((1, 1, 128), lambda h: (h, 0, 0)),
        ],
        out_specs=[
            pl.BlockSpec((2 * N, 128), lambda h: (h, 0)),
            pl.BlockSpec((N, 128), lambda h: (h, 0)),
        ],
        out_shape=[
            jax.ShapeDtypeStruct((NCHUNK * N, 128), jnp.float32),
            jax.ShapeDtypeStruct((2 * N, 128), jnp.float32),
        ],
    )(features, MB, BB, MS, BS)


_mesh = plsc.VectorSubcoreMesh(core_axis_name="c", subcore_axis_name="s")


@functools.partial(
    pl.kernel,
    mesh=_mesh,
    out_type=jax.ShapeDtypeStruct((NPASS * RANGE, 128), jnp.float32),
    scratch_types=[
        pltpu.VMEM((B,), jnp.int32),
        pltpu.VMEM((B,), jnp.int32),
        pltpu.VMEM((B,), jnp.int32),
        pltpu.VMEM((B,), jnp.int32),
        pltpu.VMEM((B, 128), jnp.float32),
        pltpu.VMEM((B, 128), jnp.float32),
        pltpu.VMEM((B, 128), jnp.float32),
        pltpu.VMEM((B, 128), jnp.float32),
        pltpu.VMEM_SHARED((NACC, 128), jnp.float32),
        pltpu.SemaphoreType.DMA,
        pltpu.SemaphoreType.DMA,
    ],
)
def _sc_edge_pass(big_tbl, small_tbl, big_idx, small_idx, dst_idx, out_acc,
                  ibig, ismall, idst, idst2, brows, srows, orows, zbuf, acc,
                  sem1, sem2):
    cid = lax.axis_index("c")
    sid = lax.axis_index("s")
    zero16 = jnp.zeros((16,), jnp.float32)

    def zrow(i, carry):
        for j in range(8):
            zbuf[i, pl.ds(16 * j, 16)] = zero16
        return carry

    lax.fori_loop(0, B, zrow, 0)

    def zero_acc():
        r0 = sid * ZROWS_A
        pltpu.sync_copy(zbuf, acc.at[pl.ds(r0, B)])
        pltpu.sync_copy(zbuf, acc.at[pl.ds(r0 + B, B)])
        pltpu.sync_copy(zbuf.at[pl.ds(0, ZROWS_A - 2 * B)],
                        acc.at[pl.ds(r0 + 2 * B, ZROWS_A - 2 * B)])

    zero_acc()
    plsc.subcore_barrier()

    tbase = sid * EPT

    for half in range(2):
        c = cid * 2 + half
        for r in range(2):

            def batch(g, carry):
                base = tbase + g * B
                pltpu.sync_copy(big_idx.at[pl.ds(c * EP + base, B)], ibig)
                pltpu.sync_copy(small_idx.at[pl.ds(cid * EP + base, B)],
                                ismall)
                pltpu.sync_copy(dst_idx.at[pl.ds(base, B)], idst)
                cp1 = pltpu.async_copy(big_tbl.at[ibig], brows, sem1)
                cp2 = pltpu.async_copy(small_tbl.at[ismall], srows, sem2)
                for k in range(B // 16):
                    d = idst[pl.ds(16 * k, 16)] - (r * RANGE)
                    okm = (d >= 0) & (d < RANGE)
                    idst2[pl.ds(16 * k, 16)] = jnp.where(okm, d, TRASH)
                cp1.wait()
                cp2.wait()

                def edge(b, ecarry):
                    for j in range(4):
                        p = brows[b, pl.ds(16 * j, 16)]
                        q = srows[b, pl.ds(64 * half + 16 * j, 16)]
                        fv = brows[b, pl.ds(64 + 16 * j, 16)]
                        a = jnp.exp(jnp.maximum(p + q, 0.0))
                        orows[b, pl.ds(16 * j, 16)] = a * fv
                        orows[b, pl.ds(64 + 16 * j, 16)] = a
                    return ecarry

                lax.fori_loop(0, B, edge, 0)
                pltpu.sync_copy(orows, acc.at[idst2], add=True)
                return carry

            lax.fori_loop(0, NB, batch, 0)
            plsc.subcore_barrier()

            fr = sid * FROWS
            oidx = 2 * c + r
            pltpu.sync_copy(acc.at[pl.ds(fr, FROWS)],
                            out_acc.at[pl.ds(oidx * RANGE + fr, FROWS)])
            if not (half == 1 and r == 1):
                plsc.subcore_barrier()
                zero_acc()
                plsc.subcore_barrier()


BR = 1024


def _stage3_body(acc_ref, w2_ref, b2_ref, out_ref):
    nfs = []
    for c in range(NCHUNK):
        a = acc_ref[c, 0]
        num, den = a[:, :64], a[:, 64:]
        den_safe = jnp.where(den > 0.0, den, 1.0)
        nfs.append(num / den_safe)
    nf = jnp.concatenate(nfs, axis=1)
    out_ref[...] = (
        jnp.dot(nf, w2_ref[...].T, preferred_element_type=jnp.float32)
        + b2_ref[...]
    )


def _stage3(acc, W2, b2):
    out = pl.pallas_call(
        _stage3_body,
        grid=(2 * RANGE // BR,),
        in_specs=[
            pl.BlockSpec((NCHUNK, 1, BR, 128),
                         lambda i: (0, lax.div(i, 5), lax.rem(i, 5), 0)),
            pl.BlockSpec((C, 2 * C), lambda i: (0, 0)),
            pl.BlockSpec((1, C), lambda i: (0, 0)),
        ],
        out_specs=pl.BlockSpec((BR, C), lambda i: (i, 0)),
        out_shape=jax.ShapeDtypeStruct((2 * RANGE, C), jnp.float32),
    )(acc, W2, b2.reshape(1, C))
    return out[:N]


def kernel(features, src_id, dst_id, W1, b1, W2, b2):
    big_tbl, small_tbl = _build_tables(features, W1, b1)

    pad = EP - E
    offs = (jnp.arange(NCHUNK, dtype=jnp.int32) * N)[:, None]
    zpad = jnp.zeros((pad,), jnp.int32)
    src_p = jnp.concatenate([src_id, zpad])
    dst_p = jnp.concatenate([dst_id, zpad])
    big_idx = (jnp.stack([src_p, src_p, dst_p, dst_p]) + offs).reshape(-1)
    small_idx = jnp.concatenate([dst_p, src_p + N])
    dst_sc = jnp.concatenate(
        [dst_id, jnp.full((pad,), N, jnp.int32)])

    acc = _sc_edge_pass(big_tbl, small_tbl, big_idx, small_idx, dst_sc)
    return _stage3(acc.reshape(NCHUNK, 2, RANGE, 128), W2, b2)


# speedup vs baseline: 2.4118x; 1.0201x over previous
import functools

import jax
import jax.numpy as jnp
from jax import lax
from jax.experimental import pallas as pl
from jax.experimental.pallas import tpu as pltpu
from jax.experimental.pallas import tpu_sc as plsc

N = 10000
E = 320000
C = 128
B = 64
NT = 16
EPT_ALIGN = NT * 16 * B
EP = ((E + EPT_ALIGN - 1) // EPT_ALIGN) * EPT_ALIGN
NR = 4
RANGE = 2560
NACC = 2688
TRASH0 = RANGE
ZROWS_A = NACC // NT
FROWS = RANGE // NT
EPB = EP // 32
BBIN = 1024


def _stage1_body(f_ref, mb_ref, bb_ref, ms_ref, bs_ref, big_ref, small_ref):
    f = f_ref[...]
    big_ref[...] = (
        jnp.dot(f, mb_ref[0].T, preferred_element_type=jnp.float32) + bb_ref[0]
    )
    small_ref[...] = (
        jnp.dot(f, ms_ref[0].T, preferred_element_type=jnp.float32) + bs_ref[0]
    )


def _build_tables(features, W1, b1):
    eye = jnp.eye(C, dtype=jnp.float32)
    zc = jnp.zeros((C,), jnp.float32)
    MB = jnp.stack([
        jnp.concatenate([W1[0:C, 0:C], eye], axis=0),
        jnp.concatenate([W1[C:, C:], eye], axis=0),
    ])
    BB = jnp.stack([
        jnp.concatenate([zc, zc]),
        jnp.concatenate([b1[C:], zc]),
    ])[:, None, :]
    MS = jnp.stack([W1[0:C, C:], W1[C:, 0:C]])
    BS = jnp.stack([b1[0:C], zc])[:, None, :]

    return pl.pallas_call(
        _stage1_body,
        grid=(2,),
        in_specs=[
            pl.BlockSpec((N, C), lambda h: (0, 0)),
            pl.BlockSpec((1, 2 * C, C), lambda h: (h, 0, 0)),
            pl.BlockSpec((1, 1, 2 * C), lambda h: (h, 0, 0)),
            pl.BlockSpec((1, C, C), lambda h: (h, 0, 0)),
            pl.BlockSpec((1, 1, C), lambda h: (h, 0, 0)),
        ],
        out_specs=[
            pl.BlockSpec((N, 2 * C), lambda h: (h, 0)),
            pl.BlockSpec((N, C), lambda h: (h, 0)),
        ],
        out_shape=[
            jax.ShapeDtypeStruct((2 * N, 2 * C), jnp.float32),
            jax.ShapeDtypeStruct((2 * N, C), jnp.float32),
        ],
    )(features, MB, BB, MS, BS)


_mesh = plsc.VectorSubcoreMesh(core_axis_name="c", subcore_axis_name="s")


@functools.partial(
    pl.kernel,
    mesh=_mesh,
    out_type=[
        jax.ShapeDtypeStruct((NR * EP,), jnp.int32),
        jax.ShapeDtypeStruct((NR * EP,), jnp.int32),
        jax.ShapeDtypeStruct((NR * 32 * 64,), jnp.int32),
    ],
    scratch_types=[
        pltpu.VMEM((BBIN,), jnp.int32),
        pltpu.VMEM((BBIN,), jnp.int32),
        pltpu.VMEM((EPB + 16,), jnp.int32),
        pltpu.VMEM((EPB + 16,), jnp.int32),
        pltpu.VMEM((EPB + 16,), jnp.int32),
        pltpu.VMEM((EPB + 16,), jnp.int32),
        pltpu.VMEM((EPB + 16,), jnp.int32),
        pltpu.VMEM((EPB + 16,), jnp.int32),
        pltpu.VMEM((EPB + 16,), jnp.int32),
        pltpu.VMEM((EPB + 16,), jnp.int32),
        pltpu.VMEM((64,), jnp.int32),
        pltpu.SemaphoreType.DMA,
    ],
    compiler_params=pltpu.CompilerParams(needs_layout_passes=False),
)
def _sc_bin(src_p, dst_p, srcL, dstL, counts,
            sstg, dstg, bs0, bd0, bs1, bd1, bs2, bd2, bs3, bd3, cstg, sem):
    cid = lax.axis_index("c")
    sid = lax.axis_index("s")
    w = cid * NT + sid
    ebase = w * EPB
    bufs = ((bs0, bd0), (bs1, bd1), (bs2, bd2), (bs3, bd3))

    def bin_batch(bi, cnts):
        cp1 = pltpu.async_copy(src_p.at[pl.ds(ebase + bi * BBIN, BBIN)],
                               sstg, sem)
        cp2 = pltpu.async_copy(dst_p.at[pl.ds(ebase + bi * BBIN, BBIN)],
                               dstg, sem)
        cp1.wait()
        cp2.wait()

        def grp(gi, cnts2):
            one16 = jnp.ones((16,), jnp.int32)
            zero16b = jnp.zeros((16,), jnp.int32)
            sv = sstg[pl.ds(16 * gi, 16)]
            dv = dstg[pl.ds(16 * gi, 16)]
            out = []
            for r in range(NR):
                if r == 0:
                    m = dv < RANGE
                elif r == NR - 1:
                    m = dv >= (NR - 1) * RANGE
                else:
                    m = (dv >= r * RANGE) & (dv < (r + 1) * RANGE)
                cs = jnp.cumsum(jnp.where(m, one16, zero16b))
                pos = cnts2[r] + cs - 1
                plsc.store_scatter(bufs[r][0], [pos], sv, mask=m)
                plsc.store_scatter(bufs[r][1], [pos], dv, mask=m)
                out.append(cnts2[r] + cs[15])
            return tuple(out)

        return lax.fori_loop(0, BBIN // 16, grp, cnts)

    cnts = lax.fori_loop(0, EPB // BBIN, bin_batch,
                         tuple(jnp.int32(0) for _ in range(NR)))

    ii = lax.iota(jnp.int32, 16)
    zero16i = jnp.zeros((16,), jnp.int32)
    for r in range(NR):
        bsr, bdr = bufs[r]
        cnt = cnts[r]
        target = lax.div(cnt + 2 * B - 1, 2 * B) * (2 * B)
        pg = lax.div(target - cnt + 15, 16)
        padv = (r * RANGE + RANGE + ii) if r < NR - 1 else (N + ii)

        def padbody(i, carry, bsr=bsr, bdr=bdr, cnt=cnt, padv=padv):
            pos = cnt + 16 * i + ii
            plsc.store_scatter(bdr, [pos], padv)
            plsc.store_scatter(bsr, [pos], zero16i)
            return carry

        lax.fori_loop(0, pg, padbody, 0)
        pltpu.sync_copy(bsr.at[pl.ds(0, EPB)],
                        srcL.at[pl.ds(r * EP + ebase, EPB)])
        pltpu.sync_copy(bdr.at[pl.ds(0, EPB)],
                        dstL.at[pl.ds(r * EP + ebase, EPB)])
        npair = jnp.full((16,), lax.div(target, 2 * B), jnp.int32)
        for q in range(4):
            cstg[pl.ds(16 * q, 16)] = npair
        pltpu.sync_copy(cstg, counts.at[pl.ds((r * 32 + w) * 64, 64)])


@functools.partial(
    pl.kernel,
    mesh=_mesh,
    out_type=[
        jax.ShapeDtypeStruct((2 * NR * RANGE, C), jnp.float32),
        jax.ShapeDtypeStruct((2 * NR * RANGE, C), jnp.float32),
    ],
    scratch_types=[
        pltpu.VMEM((2, B), jnp.int32),
        pltpu.VMEM((2, B), jnp.int32),
        pltpu.VMEM((2, B), jnp.int32),
        pltpu.VMEM((2, B), jnp.int32),
        pltpu.VMEM((2, B), jnp.int32),
        pltpu.VMEM((2, B), jnp.int32),
        pltpu.VMEM((2, B, 2 * C), jnp.float32),
        pltpu.VMEM((2, B, C), jnp.float32),
        pltpu.VMEM((2, B, C), jnp.float32),
        pltpu.VMEM((2, B, C), jnp.float32),
        pltpu.VMEM((64,), jnp.int32),
        pltpu.VMEM_SHARED((NACC, C), jnp.float32),
        pltpu.VMEM_SHARED((NACC, C), jnp.float32),
        pltpu.SemaphoreType.DMA,
        pltpu.SemaphoreType.DMA,
        pltpu.SemaphoreType.DMA,
        pltpu.SemaphoreType.DMA,
        pltpu.SemaphoreType.DMA,
    ],
)
def _sc_edge_pass(big_tbl, small_tbl, srcL, dstL, counts, outA, outB,
                  sraw, draw, ibig, ismall, srow, srowS, brows, qrows,
                  orowsA, orowsB, cstg, accA, accB, sem_i, sg0, sg1,
                  ss0, ss1):
    cid = lax.axis_index("c")
    sid = lax.axis_index("s")
    sem_g = (sg0, sg1)
    sem_s = (ss0, ss1)
    cN = cid * N
    zero16 = jnp.zeros((16,), jnp.float32)

    def zero_acc():
        def zrow(i, carry):
            for j in range(C // 16):
                orowsA[0, i, pl.ds(16 * j, 16)] = zero16
            return carry

        lax.fori_loop(0, B, zrow, 0)
        zb = orowsA.at[0]
        r0 = sid * ZROWS_A
        for acc in (accA, accB):
            for k in range(ZROWS_A // B):
                pltpu.sync_copy(zb, acc.at[pl.ds(r0 + k * B, B)])
            rem = ZROWS_A % B
            if rem:
                pltpu.sync_copy(zb.at[pl.ds(0, rem)],
                                acc.at[pl.ds(r0 + (ZROWS_A // B) * B, rem)])

    zero_acc()
    plsc.subcore_barrier()

    def scat_wait(p):
        pltpu.make_async_copy(orowsA.at[p], accA.at[srowS.at[p]],
                              sem_s[p]).wait()
        pltpu.make_async_copy(orowsB.at[p], accB.at[srowS.at[p]],
                              sem_s[p]).wait()

    def scat_issue(p):
        pltpu.async_copy(orowsA.at[p], accA.at[srowS.at[p]], sem_s[p],
                         add=True)
        pltpu.async_copy(orowsB.at[p], accB.at[srowS.at[p]], sem_s[p],
                         add=True)

    def compute(p):
        def edge(b, ecarry):
            for j in range(4):
                pa = brows[p, b, pl.ds(16 * j, 16)]
                qa = qrows[p, b, pl.ds(16 * j, 16)]
                fa = brows[p, b, pl.ds(128 + 16 * j, 16)]
                aa = jnp.exp(jnp.maximum(pa + qa, 0.0))
                orowsA[p, b, pl.ds(16 * j, 16)] = aa * fa
                orowsA[p, b, pl.ds(64 + 16 * j, 16)] = aa
                pb = brows[p, b, pl.ds(64 + 16 * j, 16)]
                qb = qrows[p, b, pl.ds(64 + 16 * j, 16)]
                fb = brows[p, b, pl.ds(192 + 16 * j, 16)]
                ab = jnp.exp(jnp.maximum(pb + qb, 0.0))
                orowsB[p, b, pl.ds(16 * j, 16)] = ab * fb
                orowsB[p, b, pl.ds(64 + 16 * j, 16)] = ab
            return ecarry

        lax.fori_loop(0, B, edge, 0)

    def range_body(r, carry):
        rbase = r * RANGE
        iiv = lax.iota(jnp.int32, 16)

        def prep(slot):
            for kk in range(B // 16):
                sv = sraw[slot, pl.ds(16 * kk, 16)]
                dv = draw[slot, pl.ds(16 * kk, 16)]
                ibig[slot, pl.ds(16 * kk, 16)] = jnp.minimum(
                    sv + (dv - sv) * cid + cN, 2 * N - 1)
                ismall[slot, pl.ds(16 * kk, 16)] = jnp.minimum(
                    dv + (sv - dv) * cid + cN, 2 * N - 1)
                row = dv - rbase
                srow[slot, pl.ds(16 * kk, 16)] = jnp.where(
                    row < RANGE, row, TRASH0 + 16 * kk + iiv)

        for wseg in (0, NT):
            w = sid + wseg
            segbase = r * EP + w * EPB
            pltpu.sync_copy(counts.at[pl.ds((r * 32 + w) * 64, 64)], cstg)
            np2 = cstg[pl.ds(0, 16)][0]
            nbatch = np2 * 2

            def iissue(x, slot):
                pltpu.async_copy(srcL.at[pl.ds(segbase + x * B, B)],
                                 sraw.at[slot], sem_i)
                pltpu.async_copy(dstL.at[pl.ds(segbase + x * B, B)],
                                 draw.at[slot], sem_i)

            def iwait(slot):
                pltpu.make_async_copy(srcL.at[pl.ds(segbase, B)],
                                      sraw.at[slot], sem_i).wait()
                pltpu.make_async_copy(dstL.at[pl.ds(segbase, B)],
                                      draw.at[slot], sem_i).wait()

            def gissue(slot):
                pltpu.async_copy(big_tbl.at[ibig.at[slot]],
                                 brows.at[slot], sem_g[slot])
                pltpu.async_copy(small_tbl.at[ismall.at[slot]],
                                 qrows.at[slot], sem_g[slot])

            def gwait(slot):
                pltpu.make_async_copy(big_tbl.at[ibig.at[slot]],
                                      brows.at[slot], sem_g[slot]).wait()
                pltpu.make_async_copy(small_tbl.at[ismall.at[slot]],
                                      qrows.at[slot], sem_g[slot]).wait()

            @pl.when(np2 > 0)
            def _():
                for k in range(2):
                    iissue(k, k)
                for k in range(2):
                    iwait(k)
                    prep(k)
                    gissue(k)

                    @pl.when(k + 2 < nbatch)
                    def _(k=k):
                        iissue(k + 2, k)

            def pairbody(t, carry2):
                b0 = 2 * t
                for j in range(2):
                    b = b0 + j
                    p = j
                    gwait(p)

                    @pl.when(t > 0)
                    def _():
                        scat_wait(p)

                    compute(p)
                    for kk in range(B // 16):
                        srowS[p, pl.ds(16 * kk, 16)] = (
                            srow[p, pl.ds(16 * kk, 16)])
                    scat_issue(p)

                    @pl.when(b + 2 < nbatch)
                    def _():
                        iwait(p)
                        prep(p)
                        gissue(p)

                        @pl.when(b + 4 < nbatch)
                        def _():
                            iissue(b + 4, p)
                return carry2

            lax.fori_loop(0, np2, pairbody, 0)

            @pl.when(np2 > 0)
            def _():
                scat_wait(0)
                scat_wait(1)

        plsc.subcore_barrier()
        fr = sid * FROWS
        oidx = cid * NR + r
        pltpu.sync_copy(accA.at[pl.ds(fr, FROWS)],
                        outA.at[pl.ds(oidx * RANGE + fr, FROWS)])
        pltpu.sync_copy(accB.at[pl.ds(fr, FROWS)],
                        outB.at[pl.ds(oidx * RANGE + fr, FROWS)])
        plsc.subcore_barrier()
        zero_acc()
        plsc.subcore_barrier()
        return carry

    lax.fori_loop(0, NR, range_body, 0)


BR = 512


def _stage3_body(accA_ref, accB_ref, w2_ref, b2_ref, out_ref):
    nfs = []
    for core in range(2):
        for a_ref in (accA_ref, accB_ref):
            a = a_ref[core, 0]
            num = a[:, :64]
            den = a[:, 64:]
            den_safe = jnp.where(den > 0.0, den, 1.0)
            nfs.append(num / den_safe)
    nf = jnp.concatenate(nfs, axis=1)
    out_ref[...] = (
        jnp.dot(nf, w2_ref[...].T, preferred_element_type=jnp.float32)
        + b2_ref[...]
    )


def _stage3(accA, accB, W2, b2):
    aspec = pl.BlockSpec((2, 1, BR, C),
                         lambda i: (0, lax.div(i, 5), lax.rem(i, 5), 0))
    out = pl.pallas_call(
        _stage3_body,
        grid=(NR * RANGE // BR,),
        in_specs=[
            aspec,
            aspec,
            pl.BlockSpec((C, 2 * C), lambda i: (0, 0)),
            pl.BlockSpec((1, C), lambda i: (0, 0)),
        ],
        out_specs=pl.BlockSpec((BR, C), lambda i: (i, 0)),
        out_shape=jax.ShapeDtypeStruct((NR * RANGE, C), jnp.float32),
    )(accA, accB, W2, b2.reshape(1, C))
    return out[:N]


def kernel(features, src_id, dst_id, W1, b1, W2, b2):
    big_tbl, small_tbl = _build_tables(features, W1, b1)

    pad = EP - E
    zpad = jnp.zeros((pad,), jnp.int32)
    src_p = jnp.concatenate([src_id, zpad])
    dst_p = jnp.concatenate(
        [dst_id, N + (jnp.arange(pad, dtype=jnp.int32) % 240)])

    srcL, dstL, counts = _sc_bin(src_p, dst_p)
    accA, accB = _sc_edge_pass(big_tbl, small_tbl, srcL, dstL, counts)
    return _stage3(accA.reshape(2, NR, RANGE, C),
                   accB.reshape(2, NR, RANGE, C), W2, b2)


# speedup vs baseline: 3.5853x; 1.4865x over previous
import functools

import jax
import jax.numpy as jnp
from jax import lax
from jax.experimental import pallas as pl
from jax.experimental.pallas import tpu as pltpu
from jax.experimental.pallas import tpu_sc as plsc

N = 10000
E = 320000
C = 128
NCHUNK = 4
B = 64
NT = 16
SB = 16
IBLK = SB * B
EPT_ALIGN = NT * IBLK
EP = ((E + EPT_ALIGN - 1) // EPT_ALIGN) * EPT_ALIGN
EPT = EP // NT
NB = EPT // B
NSB = NB // SB
NSB2 = NSB // 2
RANGE = 5120
NACC = 5248
TRASH0 = RANGE
ZROWS_A = NACC // NT
FROWS = RANGE // NT
NPASS = 8


def _stage1_body(f_ref, mb_ref, bb_ref, ms_ref, bs_ref, big_ref, small_ref):
    f = f_ref[...]
    big_ref[:N, :] = (
        jnp.dot(f, mb_ref[0].T, preferred_element_type=jnp.float32) + bb_ref[0]
    )
    big_ref[N:, :] = (
        jnp.dot(f, mb_ref[1].T, preferred_element_type=jnp.float32) + bb_ref[1]
    )
    small_ref[...] = (
        jnp.dot(f, ms_ref[0].T, preferred_element_type=jnp.float32) + bs_ref[0]
    )


def _build_tables(features, W1, b1):
    eye = jnp.eye(64, dtype=jnp.float32)
    zero = jnp.zeros((64, 64), jnp.float32)
    sel = [jnp.concatenate([eye, zero], axis=1),
           jnp.concatenate([zero, eye], axis=1)]
    mbig, bb, msm, bs = [], [], [], []
    for c in range(NCHUNK):
        rows = W1[64 * c:64 * (c + 1), :]
        b1c = b1[64 * c:64 * (c + 1)]
        if c < 2:
            wb, ws = rows[:, :C], rows[:, C:]
            bb.append(jnp.zeros((128,), jnp.float32))
            bs.append(b1c)
        else:
            wb, ws = rows[:, C:], rows[:, :C]
            bb.append(jnp.concatenate([b1c, jnp.zeros((64,), jnp.float32)]))
            bs.append(jnp.zeros((64,), jnp.float32))
        mbig.append(jnp.concatenate([wb, sel[c % 2]], axis=0))
        msm.append(ws)
    MB = jnp.stack(mbig)
    BB = jnp.stack(bb)[:, None, :]
    MS = jnp.stack([W1[0:128, C:], W1[C:, 0:C]])
    BS = jnp.stack([b1[0:C], jnp.zeros((C,), jnp.float32)])[:, None, :]

    return pl.pallas_call(
        _stage1_body,
        grid=(2,),
        in_specs=[
            pl.BlockSpec((N, C), lambda h: (0, 0)),
            pl.BlockSpec((2, 128, 128), lambda h: (h, 0, 0)),
            pl.BlockSpec((2, 1, 128), lambda h: (h, 0, 0)),
            pl.BlockSpec((1, 128, 128), lambda h: (h, 0, 0)),
            pl.BlockSpec((1, 1, 128), lambda h: (h, 0, 0)),
        ],
        out_specs=[
            pl.BlockSpec((2 * N, 128), lambda h: (h, 0)),
            pl.BlockSpec((N, 128), lambda h: (h, 0)),
        ],
        out_shape=[
            jax.ShapeDtypeStruct((NCHUNK * N, 128), jnp.float32),
            jax.ShapeDtypeStruct((2 * N, 128), jnp.float32),
        ],
    )(features, MB, BB, MS, BS)


_mesh = plsc.VectorSubcoreMesh(core_axis_name="c", subcore_axis_name="s")

EPB = EP // 32
BBIN = 1024


@functools.partial(
    pl.kernel,
    mesh=_mesh,
    out_type=[
        jax.ShapeDtypeStruct((2 * EP,), jnp.int32),
        jax.ShapeDtypeStruct((2 * EP,), jnp.int32),
        jax.ShapeDtypeStruct((4096,), jnp.int32),
    ],
    scratch_types=[
        pltpu.VMEM((BBIN,), jnp.int32),
        pltpu.VMEM((BBIN,), jnp.int32),
        pltpu.VMEM((EPB + 16,), jnp.int32),
        pltpu.VMEM((EPB + 16,), jnp.int32),
        pltpu.VMEM((EPB + 16,), jnp.int32),
        pltpu.VMEM((EPB + 16,), jnp.int32),
        pltpu.VMEM((64,), jnp.int32),
        pltpu.SemaphoreType.DMA,
    ],
    compiler_params=pltpu.CompilerParams(needs_layout_passes=False),
)
def _sc_bin(src_p, dst_p, srcL, dstL, counts,
            sstg, dstg, bs0, bd0, bs1, bd1, cstg, sem):
    cid = lax.axis_index("c")
    sid = lax.axis_index("s")
    w = cid * NT + sid
    ebase = w * EPB

    def bin_batch(bi, cnts):
        cp1 = pltpu.async_copy(src_p.at[pl.ds(ebase + bi * BBIN, BBIN)],
                               sstg, sem)
        cp2 = pltpu.async_copy(dst_p.at[pl.ds(ebase + bi * BBIN, BBIN)],
                               dstg, sem)
        cp1.wait()
        cp2.wait()

        def grp(gi, cnts2):
            cc0, cc1 = cnts2
            one16 = jnp.ones((16,), jnp.int32)
            zero16b = jnp.zeros((16,), jnp.int32)
            sv = sstg[pl.ds(16 * gi, 16)]
            dv = dstg[pl.ds(16 * gi, 16)]
            m0 = dv < RANGE
            m1 = dv >= RANGE
            cs0 = jnp.cumsum(jnp.where(m0, one16, zero16b))
            cs1 = jnp.cumsum(jnp.where(m1, one16, zero16b))
            plsc.store_scatter(bs0, [cc0 + cs0 - 1], sv, mask=m0)
            plsc.store_scatter(bd0, [cc0 + cs0 - 1], dv, mask=m0)
            plsc.store_scatter(bs1, [cc1 + cs1 - 1], sv, mask=m1)
            plsc.store_scatter(bd1, [cc1 + cs1 - 1], dv, mask=m1)
            n0 = cs0[15]
            return (cc0 + n0, cc1 + (16 - n0))

        return lax.fori_loop(0, BBIN // 16, grp, cnts)

    cnt0, cnt1 = lax.fori_loop(0, EPB // BBIN, bin_batch,
                               (jnp.int32(0), jnp.int32(0)))

    ii = lax.iota(jnp.int32, 16)
    zero16i = jnp.zeros((16,), jnp.int32)
    for r, bsr, bdr, cnt in ((0, bs0, bd0, cnt0), (1, bs1, bd1, cnt1)):
        target = lax.div(cnt + 4 * B - 1, 4 * B) * (4 * B)
        pg = lax.div(target - cnt + 15, 16)
        padv = (RANGE + ii) if r == 0 else (N + ii)

        def padbody(i, carry, bsr=bsr, bdr=bdr, cnt=cnt, padv=padv):
            pos = cnt + 16 * i + ii
            plsc.store_scatter(bdr, [pos], padv)
            plsc.store_scatter(bsr, [pos], zero16i)
            return carry

        lax.fori_loop(0, pg, padbody, 0)
        pltpu.sync_copy(bsr.at[pl.ds(0, EPB)],
                        srcL.at[pl.ds(r * EP + ebase, EPB)])
        pltpu.sync_copy(bdr.at[pl.ds(0, EPB)],
                        dstL.at[pl.ds(r * EP + ebase, EPB)])
        npair = jnp.full((16,), lax.div(target, 4 * B), jnp.int32)
        for q in range(4):
            cstg[pl.ds(16 * q, 16)] = npair
        pltpu.sync_copy(cstg, counts.at[pl.ds((r * 32 + w) * 64, 64)])


@functools.partial(
    pl.kernel,
    mesh=_mesh,
    out_type=jax.ShapeDtypeStruct((NPASS * RANGE, 128), jnp.float32),
    scratch_types=[
        pltpu.VMEM((4, B), jnp.int32),
        pltpu.VMEM((4, B), jnp.int32),
        pltpu.VMEM((4, B), jnp.int32),
        pltpu.VMEM((4, B), jnp.int32),
        pltpu.VMEM((4, B), jnp.int32),
        pltpu.VMEM((2, B), jnp.int32),
        pltpu.VMEM((4, B, 128), jnp.float32),
        pltpu.VMEM((4, B, 128), jnp.float32),
        pltpu.VMEM((2, B, 128), jnp.float32),
        pltpu.VMEM((64,), jnp.int32),
        pltpu.VMEM_SHARED((NACC, 128), jnp.float32),
        pltpu.SemaphoreType.DMA,
        pltpu.SemaphoreType.DMA,
        pltpu.SemaphoreType.DMA,
        pltpu.SemaphoreType.DMA,
        pltpu.SemaphoreType.DMA,
        pltpu.SemaphoreType.DMA,
        pltpu.SemaphoreType.DMA,
    ],
)
def _sc_edge_pass(big_tbl, small_tbl, srcL, dstL, counts, out_acc,
                  sraw, draw, ibig, ismall, srow, srowS, brows, qrows, orows,
                  cstg, acc, sem_i, sg0, sg1, sg2, sg3, ss0, ss1):
    cid = lax.axis_index("c")
    sid = lax.axis_index("s")
    sem_g = (sg0, sg1, sg2, sg3)
    sem_s = (ss0, ss1)
    zero16 = jnp.zeros((16,), jnp.float32)

    def zero_acc():
        def zrow(i, carry):
            for j in range(8):
                orows[0, i, pl.ds(16 * j, 16)] = zero16
            return carry

        lax.fori_loop(0, B, zrow, 0)
        zb = orows.at[0]
        r0 = sid * ZROWS_A
        for k in range(ZROWS_A // B):
            pltpu.sync_copy(zb, acc.at[pl.ds(r0 + k * B, B)])
        rem = ZROWS_A % B
        if rem:
            pltpu.sync_copy(zb.at[pl.ds(0, rem)],
                            acc.at[pl.ds(r0 + (ZROWS_A // B) * B, rem)])

    zero_acc()
    plsc.subcore_barrier()

    for half in range(2):
        c = cid * 2 + half
        cN = c * N

        def scat_wait(p):
            pltpu.make_async_copy(orows.at[p], acc.at[srowS.at[p]],
                                  sem_s[p]).wait()

        def scat_issue(p):
            pltpu.async_copy(orows.at[p], acc.at[srowS.at[p]], sem_s[p],
                             add=True)

        def compute(slot, p):
            def edge(b, ecarry):
                for j in range(4):
                    pv = brows[slot, b, pl.ds(16 * j, 16)]
                    qv = qrows[slot, b, pl.ds(64 * half + 16 * j, 16)]
                    fv = brows[slot, b, pl.ds(64 + 16 * j, 16)]
                    a = jnp.exp(jnp.maximum(pv + qv, 0.0))
                    orows[p, b, pl.ds(16 * j, 16)] = a * fv
                    orows[p, b, pl.ds(64 + 16 * j, 16)] = a
                return ecarry

            lax.fori_loop(0, B, edge, 0)

        def range_body(r, carry):
            rbase = r * RANGE
            iiv = lax.iota(jnp.int32, 16)

            def prep(slot):
                for kk in range(B // 16):
                    sv = sraw[slot, pl.ds(16 * kk, 16)]
                    dv = draw[slot, pl.ds(16 * kk, 16)]
                    ibig[slot, pl.ds(16 * kk, 16)] = jnp.minimum(
                        sv + (dv - sv) * cid + cN, 4 * N - 1)
                    ismall[slot, pl.ds(16 * kk, 16)] = (
                        dv + (sv + N - dv) * cid)
                    row = dv - rbase
                    srow[slot, pl.ds(16 * kk, 16)] = jnp.where(
                        row < RANGE, row, TRASH0 + 16 * kk + iiv)

            for wseg in (0, NT):
                w = sid + wseg
                segbase = r * EP + w * EPB
                pltpu.sync_copy(counts.at[pl.ds((r * 32 + w) * 64, 64)],
                                cstg)
                np4 = cstg[pl.ds(0, 16)][0]
                nbatch = np4 * 4

                def iissue(x, slot):
                    pltpu.async_copy(srcL.at[pl.ds(segbase + x * B, B)],
                                     sraw.at[slot], sem_i)
                    pltpu.async_copy(dstL.at[pl.ds(segbase + x * B, B)],
                                     draw.at[slot], sem_i)

                def iwait(slot):
                    pltpu.make_async_copy(srcL.at[pl.ds(segbase, B)],
                                          sraw.at[slot], sem_i).wait()
                    pltpu.make_async_copy(dstL.at[pl.ds(segbase, B)],
                                          draw.at[slot], sem_i).wait()

                def gissue(slot):
                    pltpu.async_copy(big_tbl.at[ibig.at[slot]],
                                     brows.at[slot], sem_g[slot])
                    pltpu.async_copy(small_tbl.at[ismall.at[slot]],
                                     qrows.at[slot], sem_g[slot])

                def gwait(slot):
                    pltpu.make_async_copy(big_tbl.at[ibig.at[slot]],
                                          brows.at[slot],
                                          sem_g[slot]).wait()
                    pltpu.make_async_copy(small_tbl.at[ismall.at[slot]],
                                          qrows.at[slot],
                                          sem_g[slot]).wait()

                @pl.when(np4 > 0)
                def _():
                    for k in range(4):
                        @pl.when(k < nbatch)
                        def _(k=k):
                            iissue(k, k)
                    for k in range(4):
                        @pl.when(k < nbatch)
                        def _(k=k):
                            iwait(k)
                            prep(k)
                            gissue(k)

                            @pl.when(k + 4 < nbatch)
                            def _():
                                iissue(k + 4, k)

                def quad(t, carry2):
                    b0 = 4 * t
                    for j in range(4):
                        b = b0 + j
                        p = j % 2
                        gwait(j)
                        if j >= 2:
                            scat_wait(p)
                        else:
                            @pl.when(t > 0)
                            def _():
                                scat_wait(p)
                        compute(j, p)
                        for kk in range(B // 16):
                            srowS[p, pl.ds(16 * kk, 16)] = (
                                srow[j, pl.ds(16 * kk, 16)])
                        scat_issue(p)

                        @pl.when(b + 4 < nbatch)
                        def _():
                            iwait(j)
                            prep(j)
                            gissue(j)

                            @pl.when(b + 8 < nbatch)
                            def _():
                                iissue(b + 8, j)
                    return carry2

                lax.fori_loop(0, np4, quad, 0)

                @pl.when(np4 > 0)
                def _():
                    scat_wait(0)
                    scat_wait(1)

            plsc.subcore_barrier()
            fr = sid * FROWS
            oidx = 2 * c + r
            pltpu.sync_copy(acc.at[pl.ds(fr, FROWS)],
                            out_acc.at[pl.ds(oidx * RANGE + fr, FROWS)])
            plsc.subcore_barrier()
            zero_acc()
            plsc.subcore_barrier()
            return carry

        lax.fori_loop(0, 2, range_body, 0)


BR = 1024


def _stage3_body(acc_ref, w2_ref, b2_ref, out_ref):
    nfs = []
    for c in range(NCHUNK):
        a = acc_ref[c, 0]
        num, den = a[:, :64], a[:, 64:]
        den_safe = jnp.where(den > 0.0, den, 1.0)
        nfs.append(num / den_safe)
    nf = jnp.concatenate(nfs, axis=1)
    out_ref[...] = (
        jnp.dot(nf, w2_ref[...].T, preferred_element_type=jnp.float32)
        + b2_ref[...]
    )


def _stage3(acc, W2, b2):
    out = pl.pallas_call(
        _stage3_body,
        grid=(2 * RANGE // BR,),
        in_specs=[
            pl.BlockSpec((NCHUNK, 1, BR, 128),
                         lambda i: (0, lax.div(i, 5), lax.rem(i, 5), 0)),
            pl.BlockSpec((C, 2 * C), lambda i: (0, 0)),
            pl.BlockSpec((1, C), lambda i: (0, 0)),
        ],
        out_specs=pl.BlockSpec((BR, C), lambda i: (i, 0)),
        out_shape=jax.ShapeDtypeStruct((2 * RANGE, C), jnp.float32),
    )(acc, W2, b2.reshape(1, C))
    return out[:N]


def kernel(features, src_id, dst_id, W1, b1, W2, b2):
    big_tbl, small_tbl = _build_tables(features, W1, b1)

    pad = EP - E
    zpad = jnp.zeros((pad,), jnp.int32)
    src_p = jnp.concatenate([src_id, zpad])
    dst_p = jnp.concatenate(
        [dst_id, N + (jnp.arange(pad, dtype=jnp.int32) % 240)])

    srcL, dstL, counts = _sc_bin(src_p, dst_p)
    acc = _sc_edge_pass(big_tbl, small_tbl, srcL, dstL, counts)
    return _stage3(acc.reshape(NCHUNK, 2, RANGE, 128), W2, b2)
